# baseline (speedup 1.0000x reference)
import functools

import jax
import jax.numpy as jnp
from jax import lax
from jax.experimental import pallas as pl
from jax.experimental.pallas import tpu as pltpu
from jax.experimental.pallas import tpu_sc as plsc

T, D, DFF, E, K = 2048, 1024, 2048, 8, 2
BM = 256
NB = T * K // BM + E
S = NB * BM
CHUNK = 128
NCHUNK = T // CHUNK
NBE = 32


def _route_body(x_ref, gw_ref, gb_ref, vals_ref, dest_ref, be_ref,
                rix_ref, pn_ref, R_s, O0_s, O1_s):
    ltri = (lax.broadcasted_iota(jnp.int32, (CHUNK, CHUNK), 0) >
            lax.broadcasted_iota(jnp.int32, (CHUNK, CHUNK), 1)
            ).astype(jnp.float32)
    e_iota = lax.broadcasted_iota(jnp.int32, (CHUNK, E), 1)

    def pass1(i, carry):
        xc = x_ref[pl.ds(i * CHUNK, CHUNK), :]
        logits = jnp.dot(xc, gw_ref[:, :],
                         preferred_element_type=jnp.float32) + gb_ref[0, :]
        m = jnp.max(logits, axis=1, keepdims=True)
        p = jnp.exp(logits - m)
        p = p / jnp.sum(p, axis=1, keepdims=True)
        mx0 = jnp.max(p, axis=1, keepdims=True)
        a0 = jnp.min(jnp.where(p == mx0, e_iota, E), axis=1)
        o0 = (e_iota == a0[:, None]).astype(jnp.float32)
        v0 = jnp.sum(p * o0, axis=1, keepdims=True)
        pm = jnp.where(o0 > 0, -jnp.inf, p)
        mx1 = jnp.max(pm, axis=1, keepdims=True)
        a1 = jnp.min(jnp.where(pm == mx1, e_iota, E), axis=1)
        o1 = (e_iota == a1[:, None]).astype(jnp.float32)
        v1 = jnp.sum(p * o1, axis=1, keepdims=True)
        vals_ref[pl.ds(i * CHUNK, CHUNK), :] = jnp.concatenate([v0, v1], 1)
        osum = o0 + o1
        R_s[pl.ds(i * CHUNK, CHUNK), :] = jnp.dot(
            ltri, osum, preferred_element_type=jnp.float32) + carry
        O0_s[pl.ds(i * CHUNK, CHUNK), :] = o0
        O1_s[pl.ds(i * CHUNK, CHUNK), :] = o1
        return carry + jnp.sum(osum, axis=0, keepdims=True)

    counts = lax.fori_loop(0, NCHUNK, pass1, jnp.zeros((1, E), jnp.float32))
    pc = jnp.floor((counts + (BM - 1)) * (1.0 / BM)) * BM
    mtri = (lax.broadcasted_iota(jnp.int32, (E, E), 0) <
            lax.broadcasted_iota(jnp.int32, (E, E), 1)).astype(jnp.float32)
    po = jnp.dot(pc, mtri, preferred_element_type=jnp.float32)
    sb = (po * (1.0 / BM)).astype(jnp.int32)
    b_iota = lax.broadcasted_iota(jnp.int32, (NBE, E), 0)
    be_ref[:] = jnp.sum((b_iota >= sb).astype(jnp.int32), axis=1) - 1
    ne = (counts > 0.0).astype(jnp.int32)
    rix_ref[:] = jnp.sum(((b_iota >= sb) * ne), axis=1) - 1
    big = NBE + 1
    cand = jnp.where((sb > b_iota) & (ne > 0), sb, big)
    nxt = jnp.min(cand, axis=1, keepdims=True)
    pnv = jnp.sum((nxt >= sb).astype(jnp.int32), axis=1, keepdims=True) - 1
    pn_ref[:] = jnp.where(nxt < big, pnv, -1)[:, 0]

    def pass2(i, _):
        sl = pl.ds(i * CHUNK, CHUNK)
        r, o0, o1 = R_s[sl, :], O0_s[sl, :], O1_s[sl, :]
        d0 = jnp.sum(o0 * (po + r), axis=1, keepdims=True)
        d1 = jnp.sum(o1 * (po + r + o0), axis=1, keepdims=True)
        dest_ref[sl, :] = jnp.concatenate([d0, d1], 1).astype(jnp.int32)
        return 0

    lax.fori_loop(0, NCHUNK, pass2, 0)


_route = pl.pallas_call(
    _route_body,
    out_shape=(
        jax.ShapeDtypeStruct((T, K), jnp.float32),
        jax.ShapeDtypeStruct((T, K), jnp.int32),
        jax.ShapeDtypeStruct((NBE,), jnp.int32),
        jax.ShapeDtypeStruct((NBE,), jnp.int32),
        jax.ShapeDtypeStruct((NBE,), jnp.int32),
    ),
    scratch_shapes=[
        pltpu.VMEM((T, E), jnp.float32),
        pltpu.VMEM((T, E), jnp.float32),
        pltpu.VMEM((T, E), jnp.float32),
    ],
)


_NW = 32


@functools.cache
def _sc_mesh():
    return plsc.VectorSubcoreMesh(core_axis_name="c", subcore_axis_name="s")


@functools.cache
def _make_scatter_src():
    @functools.partial(
        pl.kernel,
        out_type=jax.ShapeDtypeStruct((S,), jnp.int32),
        mesh=_sc_mesh(),
        scratch_types=[
            pltpu.VMEM((T * K,), jnp.int32),
            pltpu.VMEM((S,), jnp.int32),
        ],
        compiler_params=pltpu.CompilerParams(needs_layout_passes=False),
    )
    def _scatter_src(dest_hbm, src_hbm, dest_v, src_v):
        wid = lax.axis_index("s") * 2 + lax.axis_index("c")

        @pl.when(wid == 0)
        def _():
            pltpu.sync_copy(dest_hbm, dest_v)
            lane = lax.iota(jnp.int32, 16)

            def initb(i, _):
                src_v[pl.ds(i * 16, 16)] = (i * 16 + lane) & (T - 1)
                return 0

            lax.fori_loop(0, S // 16, initb, 0)

            def scat(i, _):
                idx = dest_v[pl.ds(i * 16, 16)]
                tok = (i * 16 + lane) & (T - 1)
                plsc.store_scatter(src_v, [idx], tok)
                return 0

            lax.fori_loop(0, T * K // 16, scat, 0)
            pltpu.sync_copy(src_v, src_hbm)

    return _scatter_src


@functools.cache
def _make_gather(n_rows):
    b_per_w = n_rows // _NW
    gchunk = 64
    nch = b_per_w // gchunk

    @functools.partial(
        pl.kernel,
        out_type=jax.ShapeDtypeStruct((n_rows, D), jnp.float32),
        mesh=_sc_mesh(),
        scratch_types=[
            pltpu.VMEM((b_per_w,), jnp.int32),
            pltpu.VMEM((gchunk, D), jnp.float32),
            pltpu.SemaphoreType.DMA,
        ],
        compiler_params=pltpu.CompilerParams(needs_layout_passes=False),
    )
    def _gather(table_hbm, idx_hbm, out_hbm, idx_v, rows_v, sem):
        wid = lax.axis_index("s") * 2 + lax.axis_index("c")
        base = wid * b_per_w
        pltpu.sync_copy(idx_hbm.at[pl.ds(base, b_per_w)], idx_v)
        for c in range(nch):
            pltpu.async_copy(
                table_hbm.at[idx_v.at[pl.ds(c * gchunk, gchunk)]],
                rows_v, sem).wait()
            pltpu.sync_copy(
                rows_v, out_hbm.at[pl.ds(base + c * gchunk, gchunk)])

    return _gather


def _ffn_body(be_ref, rix_ref, pn_ref, xs_ref, b1_ref, b2_ref,
              w1_hbm, w2_hbm, y_ref, w1buf, w2buf, sems):
    b = pl.program_id(0)
    cur = be_ref[b]
    prev = be_ref[jnp.maximum(b - 1, 0)]
    chg = jnp.logical_or(b == 0, cur != prev)
    slot = lax.rem(rix_ref[b], 2)
    pn = pn_ref[b]

    def start_pair(eidx, sl):
        pltpu.make_async_copy(
            w1_hbm.at[pl.ds(eidx, 1)], w1buf.at[pl.ds(sl, 1)],
            sems.at[sl]).start()
        pltpu.make_async_copy(
            w2_hbm.at[pl.ds(eidx, 1)], w2buf.at[pl.ds(sl, 1)],
            sems.at[sl]).start()

    @pl.when(b == 0)
    def _():
        start_pair(cur, 0)

        @pl.when(pn >= 0)
        def _():
            start_pair(pn, 1)

    @pl.when((b > 0) & chg & (pn >= 0))
    def _():
        start_pair(pn, 1 - slot)

    @pl.when(chg)
    def _():
        pltpu.make_async_copy(
            w1_hbm.at[pl.ds(cur, 1)], w1buf.at[pl.ds(slot, 1)],
            sems.at[slot]).wait()
        pltpu.make_async_copy(
            w2_hbm.at[pl.ds(cur, 1)], w2buf.at[pl.ds(slot, 1)],
            sems.at[slot]).wait()

    h = jnp.maximum(
        jnp.dot(xs_ref[:, :], w1buf[slot],
                preferred_element_type=jnp.float32) + b1_ref[0, 0, :], 0.0)
    y_ref[:, :] = jnp.dot(
        h, w2buf[slot], preferred_element_type=jnp.float32) + b2_ref[0, 0, :]


_ffn = pl.pallas_call(
    _ffn_body,
    grid_spec=pltpu.PrefetchScalarGridSpec(
        num_scalar_prefetch=3,
        grid=(NB,),
        in_specs=[
            pl.BlockSpec((BM, D), lambda b, be, rix, pn: (b, 0)),
            pl.BlockSpec((1, 1, DFF), lambda b, be, rix, pn: (be[b], 0, 0)),
            pl.BlockSpec((1, 1, D), lambda b, be, rix, pn: (be[b], 0, 0)),
            pl.BlockSpec(memory_space=pl.ANY),
            pl.BlockSpec(memory_space=pl.ANY),
        ],
        out_specs=pl.BlockSpec((BM, D), lambda b, be, rix, pn: (b, 0)),
        scratch_shapes=[
            pltpu.VMEM((2, D, DFF), jnp.bfloat16),
            pltpu.VMEM((2, DFF, D), jnp.bfloat16),
            pltpu.SemaphoreType.DMA((2,)),
        ],
    ),
    out_shape=jax.ShapeDtypeStruct((S, D), jnp.float32),
    compiler_params=pltpu.CompilerParams(vmem_limit_bytes=64 * 1024 * 1024),
)


def _combine_body(x_ref, c0_ref, c1_ref, vals_ref, g_ref, b_ref, out_ref):
    v0 = vals_ref[:, 0:1]
    v1 = vals_ref[:, 1:2]
    o = x_ref[:, :] + v0 * c0_ref[:, :] + v1 * c1_ref[:, :]
    mu = jnp.mean(o, axis=1, keepdims=True)
    ctr = o - mu
    var = jnp.mean(ctr * ctr, axis=1, keepdims=True)
    out_ref[:, :] = ctr * lax.rsqrt(var + 1e-5) * g_ref[0, :] + b_ref[0, :]


_combine = pl.pallas_call(
    _combine_body,
    grid=(T // BM,),
    in_specs=[
        pl.BlockSpec((BM, D), lambda b: (b, 0)),
        pl.BlockSpec((BM, D), lambda b: (b, 0)),
        pl.BlockSpec((BM, D), lambda b: (b + T // BM, 0)),
        pl.BlockSpec((BM, K), lambda b: (b, 0)),
        pl.BlockSpec((1, D), lambda b: (0, 0)),
        pl.BlockSpec((1, D), lambda b: (0, 0)),
    ],
    out_specs=pl.BlockSpec((BM, D), lambda b: (b, 0)),
    out_shape=jax.ShapeDtypeStruct((T, D), jnp.float32),
)


def kernel(x, gate_w, gate_b, w1, b1, w2, b2, ln_g, ln_b):
    vals, dest, be, rix, pn = _route(x, gate_w, gate_b.reshape(1, E))
    destf = dest.T.reshape(T * K)
    src = _make_scatter_src()(destf)
    xs = _make_gather(S)(x, src)
    w1d, w2d, _ = lax.optimization_barrier((w1, w2, be))
    w1b = w1d.astype(jnp.bfloat16)
    w2b = w2d.astype(jnp.bfloat16)
    y = _ffn(be, rix, pn, xs, b1.reshape(E, 1, DFF), b2.reshape(E, 1, D),
             w1b, w2b)
    contrib = _make_gather(T * K)(y, destf)
    return _combine(x, contrib, contrib, vals,
                    ln_g.reshape(1, D), ln_b.reshape(1, D))


# speedup vs baseline: 1.2319x; 1.2319x over previous
import functools

import jax
import jax.numpy as jnp
from jax import lax
from jax.experimental import pallas as pl
from jax.experimental.pallas import tpu as pltpu
from jax.experimental.pallas import tpu_sc as plsc

T, D, DFF, E, K = 2048, 1024, 2048, 8, 2
BM = 256
NB = T * K // BM + E
S = NB * BM
CHUNK = 128
NCHUNK = T // CHUNK
NBE = 32


def _route_body(x_ref, gw_ref, gb_ref, vals_ref, dest_ref, be_ref,
                rix_ref, pn_ref, R_s, O0_s, O1_s):
    ltri = (lax.broadcasted_iota(jnp.int32, (CHUNK, CHUNK), 0) >
            lax.broadcasted_iota(jnp.int32, (CHUNK, CHUNK), 1)
            ).astype(jnp.float32)
    e_iota = lax.broadcasted_iota(jnp.int32, (CHUNK, E), 1)

    def pass1(i, carry):
        xc = x_ref[pl.ds(i * CHUNK, CHUNK), :]
        logits = jnp.dot(xc, gw_ref[:, :],
                         preferred_element_type=jnp.float32) + gb_ref[0, :]
        m = jnp.max(logits, axis=1, keepdims=True)
        p = jnp.exp(logits - m)
        p = p / jnp.sum(p, axis=1, keepdims=True)
        mx0 = jnp.max(p, axis=1, keepdims=True)
        a0 = jnp.min(jnp.where(p == mx0, e_iota, E), axis=1)
        o0 = (e_iota == a0[:, None]).astype(jnp.float32)
        v0 = jnp.sum(p * o0, axis=1, keepdims=True)
        pm = jnp.where(o0 > 0, -jnp.inf, p)
        mx1 = jnp.max(pm, axis=1, keepdims=True)
        a1 = jnp.min(jnp.where(pm == mx1, e_iota, E), axis=1)
        o1 = (e_iota == a1[:, None]).astype(jnp.float32)
        v1 = jnp.sum(p * o1, axis=1, keepdims=True)
        vals_ref[pl.ds(i * CHUNK, CHUNK), :] = jnp.concatenate([v0, v1], 1)
        osum = o0 + o1
        R_s[pl.ds(i * CHUNK, CHUNK), :] = jnp.dot(
            ltri, osum, preferred_element_type=jnp.float32) + carry
        O0_s[pl.ds(i * CHUNK, CHUNK), :] = o0
        O1_s[pl.ds(i * CHUNK, CHUNK), :] = o1
        return carry + jnp.sum(osum, axis=0, keepdims=True)

    counts = lax.fori_loop(0, NCHUNK, pass1, jnp.zeros((1, E), jnp.float32))
    pc = jnp.floor((counts + (BM - 1)) * (1.0 / BM)) * BM
    mtri = (lax.broadcasted_iota(jnp.int32, (E, E), 0) <
            lax.broadcasted_iota(jnp.int32, (E, E), 1)).astype(jnp.float32)
    po = jnp.dot(pc, mtri, preferred_element_type=jnp.float32)
    sb = (po * (1.0 / BM)).astype(jnp.int32)
    b_iota = lax.broadcasted_iota(jnp.int32, (NBE, E), 0)
    be_ref[:] = jnp.sum((b_iota >= sb).astype(jnp.int32), axis=1) - 1
    ne = (counts > 0.0).astype(jnp.int32)
    rix_ref[:] = jnp.sum(((b_iota >= sb) * ne), axis=1) - 1
    big = NBE + 1
    cand = jnp.where((sb > b_iota) & (ne > 0), sb, big)
    nxt = jnp.min(cand, axis=1, keepdims=True)
    pnv = jnp.sum((nxt >= sb).astype(jnp.int32), axis=1, keepdims=True) - 1
    pn_ref[:] = jnp.where(nxt < big, pnv, -1)[:, 0]

    def pass2(i, _):
        sl = pl.ds(i * CHUNK, CHUNK)
        r, o0, o1 = R_s[sl, :], O0_s[sl, :], O1_s[sl, :]
        d0 = jnp.sum(o0 * (po + r), axis=1, keepdims=True)
        d1 = jnp.sum(o1 * (po + r + o0), axis=1, keepdims=True)
        dest_ref[sl, :] = jnp.concatenate([d0, d1], 1).astype(jnp.int32)
        return 0

    lax.fori_loop(0, NCHUNK, pass2, 0)


_route = pl.pallas_call(
    _route_body,
    out_shape=(
        jax.ShapeDtypeStruct((T, K), jnp.float32),
        jax.ShapeDtypeStruct((T, K), jnp.int32),
        jax.ShapeDtypeStruct((NBE,), jnp.int32),
        jax.ShapeDtypeStruct((NBE,), jnp.int32),
        jax.ShapeDtypeStruct((NBE,), jnp.int32),
    ),
    scratch_shapes=[
        pltpu.VMEM((T, E), jnp.float32),
        pltpu.VMEM((T, E), jnp.float32),
        pltpu.VMEM((T, E), jnp.float32),
    ],
)


_NW = 32


@functools.cache
def _sc_mesh():
    return plsc.VectorSubcoreMesh(core_axis_name="c", subcore_axis_name="s")


@functools.cache
def _make_scatter_src():
    @functools.partial(
        pl.kernel,
        out_type=jax.ShapeDtypeStruct((S,), jnp.int32),
        mesh=_sc_mesh(),
        scratch_types=[
            pltpu.VMEM((T * K,), jnp.int32),
            pltpu.VMEM((S,), jnp.int32),
        ],
        compiler_params=pltpu.CompilerParams(needs_layout_passes=False),
    )
    def _scatter_src(dest_hbm, src_hbm, dest_v, src_v):
        wid = lax.axis_index("s") * 2 + lax.axis_index("c")

        @pl.when(wid == 0)
        def _():
            pltpu.sync_copy(dest_hbm, dest_v)
            lane = lax.iota(jnp.int32, 16)

            def initb(i, _):
                src_v[pl.ds(i * 16, 16)] = (i * 16 + lane) & (T - 1)
                return 0

            lax.fori_loop(0, S // 16, initb, 0)

            def scat(i, _):
                idx = dest_v[pl.ds(i * 16, 16)]
                tok = (i * 16 + lane) & (T - 1)
                plsc.store_scatter(src_v, [idx], tok)
                return 0

            lax.fori_loop(0, T * K // 16, scat, 0)
            pltpu.sync_copy(src_v, src_hbm)

    return _scatter_src


@functools.cache
def _make_gather(n_rows):
    b_per_w = n_rows // _NW
    gchunk = 64
    nch = b_per_w // gchunk

    @functools.partial(
        pl.kernel,
        out_type=jax.ShapeDtypeStruct((n_rows, D), jnp.float32),
        mesh=_sc_mesh(),
        scratch_types=[
            pltpu.VMEM((b_per_w,), jnp.int32),
            pltpu.VMEM((gchunk, D), jnp.float32),
            pltpu.SemaphoreType.DMA,
        ],
        compiler_params=pltpu.CompilerParams(needs_layout_passes=False),
    )
    def _gather(table_hbm, idx_hbm, out_hbm, idx_v, rows_v, sem):
        wid = lax.axis_index("s") * 2 + lax.axis_index("c")
        base = wid * b_per_w
        pltpu.sync_copy(idx_hbm.at[pl.ds(base, b_per_w)], idx_v)
        for c in range(nch):
            pltpu.async_copy(
                table_hbm.at[idx_v.at[pl.ds(c * gchunk, gchunk)]],
                rows_v, sem).wait()
            pltpu.sync_copy(
                rows_v, out_hbm.at[pl.ds(base + c * gchunk, gchunk)])

    return _gather


def _ffn_body(be_ref, rix_ref, pn_ref, xs_ref, b1_ref, b2_ref,
              w1_hbm, w2_hbm, y_ref, w1buf, w2buf, sems):
    b = pl.program_id(0)
    cur = be_ref[b]
    prev = be_ref[jnp.maximum(b - 1, 0)]
    chg = jnp.logical_or(b == 0, cur != prev)
    slot = lax.rem(rix_ref[b], 2)
    pn = pn_ref[b]

    def _pair_copies(eidx, sl):
        h1, h2 = D // 2, DFF // 2
        return [
            pltpu.make_async_copy(
                w1_hbm.at[pl.ds(eidx, 1), pl.ds(0, h1)],
                w1buf.at[pl.ds(sl, 1), pl.ds(0, h1)], sems.at[sl]),
            pltpu.make_async_copy(
                w1_hbm.at[pl.ds(eidx, 1), pl.ds(h1, h1)],
                w1buf.at[pl.ds(sl, 1), pl.ds(h1, h1)], sems.at[sl]),
            pltpu.make_async_copy(
                w2_hbm.at[pl.ds(eidx, 1), pl.ds(0, h2)],
                w2buf.at[pl.ds(sl, 1), pl.ds(0, h2)], sems.at[sl]),
            pltpu.make_async_copy(
                w2_hbm.at[pl.ds(eidx, 1), pl.ds(h2, h2)],
                w2buf.at[pl.ds(sl, 1), pl.ds(h2, h2)], sems.at[sl]),
        ]

    def start_pair(eidx, sl):
        for cp in _pair_copies(eidx, sl):
            cp.start()

    @pl.when(b == 0)
    def _():
        start_pair(cur, 0)

        @pl.when(pn >= 0)
        def _():
            start_pair(pn, 1)

    @pl.when((b > 0) & chg & (pn >= 0))
    def _():
        start_pair(pn, 1 - slot)

    @pl.when(chg)
    def _():
        for cp in _pair_copies(cur, slot):
            cp.wait()

    h = jnp.maximum(
        jnp.dot(xs_ref[:, :], w1buf[slot],
                preferred_element_type=jnp.float32) + b1_ref[0, 0, :], 0.0)
    y_ref[:, :] = jnp.dot(
        h, w2buf[slot], preferred_element_type=jnp.float32) + b2_ref[0, 0, :]


_ffn = pl.pallas_call(
    _ffn_body,
    grid_spec=pltpu.PrefetchScalarGridSpec(
        num_scalar_prefetch=3,
        grid=(NB,),
        in_specs=[
            pl.BlockSpec((BM, D), lambda b, be, rix, pn: (b, 0)),
            pl.BlockSpec((1, 1, DFF), lambda b, be, rix, pn: (be[b], 0, 0)),
            pl.BlockSpec((1, 1, D), lambda b, be, rix, pn: (be[b], 0, 0)),
            pl.BlockSpec(memory_space=pl.ANY),
            pl.BlockSpec(memory_space=pl.ANY),
        ],
        out_specs=pl.BlockSpec((BM, D), lambda b, be, rix, pn: (b, 0)),
        scratch_shapes=[
            pltpu.VMEM((2, D, DFF), jnp.float32),
            pltpu.VMEM((2, DFF, D), jnp.float32),
            pltpu.SemaphoreType.DMA((2,)),
        ],
    ),
    out_shape=jax.ShapeDtypeStruct((S, D), jnp.float32),
    compiler_params=pltpu.CompilerParams(vmem_limit_bytes=64 * 1024 * 1024),
)


def _combine_body(x_ref, c0_ref, c1_ref, vals_ref, g_ref, b_ref, out_ref):
    v0 = vals_ref[:, 0:1]
    v1 = vals_ref[:, 1:2]
    o = x_ref[:, :] + v0 * c0_ref[:, :] + v1 * c1_ref[:, :]
    mu = jnp.mean(o, axis=1, keepdims=True)
    ctr = o - mu
    var = jnp.mean(ctr * ctr, axis=1, keepdims=True)
    out_ref[:, :] = ctr * lax.rsqrt(var + 1e-5) * g_ref[0, :] + b_ref[0, :]


_combine = pl.pallas_call(
    _combine_body,
    grid=(T // BM,),
    in_specs=[
        pl.BlockSpec((BM, D), lambda b: (b, 0)),
        pl.BlockSpec((BM, D), lambda b: (b, 0)),
        pl.BlockSpec((BM, D), lambda b: (b + T // BM, 0)),
        pl.BlockSpec((BM, K), lambda b: (b, 0)),
        pl.BlockSpec((1, D), lambda b: (0, 0)),
        pl.BlockSpec((1, D), lambda b: (0, 0)),
    ],
    out_specs=pl.BlockSpec((BM, D), lambda b: (b, 0)),
    out_shape=jax.ShapeDtypeStruct((T, D), jnp.float32),
)


def kernel(x, gate_w, gate_b, w1, b1, w2, b2, ln_g, ln_b):
    vals, dest, be, rix, pn = _route(x, gate_w, gate_b.reshape(1, E))
    destf = dest.T.reshape(T * K)
    src = _make_scatter_src()(destf)
    xs = _make_gather(S)(x, src)
    y = _ffn(be, rix, pn, xs, b1.reshape(E, 1, DFF), b2.reshape(E, 1, D),
             w1, w2)
    contrib = _make_gather(T * K)(y, destf)
    return _combine(x, contrib, contrib, vals,
                    ln_g.reshape(1, D), ln_b.reshape(1, D))
